# Initial kernel scaffold; baseline (speedup 1.0000x reference)
import jax
import jax.numpy as jnp
from jax.experimental import pallas as pl


def kernel(x, edge_index, batch, W1, b1, W2, b2, W3, b3, Wlin, blin):
    raise NotImplementedError("write your pallas kernel here")


# speedup vs baseline: 7.4583x; 7.4583x over previous
import functools

import jax
import jax.numpy as jnp
from jax import lax
from jax.experimental import pallas as pl
from jax.experimental.pallas import tpu as pltpu
from jax.experimental.pallas import tpu_sc as plsc

N = 10000
E = 320000
D_IN = 128
D_H = 256
D_OUT = 128
G = 64

NP = 10240
DUMMY = N
NCORE = 2
NSUB = 16
CHUNK = 128
KB = 8
NB = 20
NCH = NB * KB
EP = NSUB * NCH * CHUNK
RPT = NP // NSUB
HALF = D_H // 2
RB = 1024


def _sc_mesh():
    return plsc.VectorSubcoreMesh(
        core_axis_name="c", subcore_axis_name="s", num_cores=NCORE,
        num_subcores=NSUB)


@functools.cache
def _make_deg_kernel():
    return functools.partial(
        pl.kernel,
        out_type=jax.ShapeDtypeStruct((NP, 128), jnp.float32),
        mesh=_sc_mesh(),
        scratch_types=[
            pltpu.VMEM((NCH, CHUNK), jnp.int32),
            pltpu.VMEM((CHUNK, 128), jnp.float32),
            pltpu.VMEM_SHARED((NP, 128), jnp.float32),
        ],
    )(_deg_body)


def _deg_body(dst_hbm, ones_hbm, zeros_hbm, deg_hbm, dst_v, ones_v, acc):
    c = lax.axis_index("c")
    s = lax.axis_index("s")
    pltpu.sync_copy(dst_hbm.at[s], dst_v)
    pltpu.sync_copy(ones_hbm, ones_v)
    pltpu.sync_copy(zeros_hbm.at[pl.ds(s * RPT, RPT)], acc.at[pl.ds(s * RPT, RPT)])
    plsc.subcore_barrier()

    def body(j, carry):
        pltpu.sync_copy(ones_v, acc.at[dst_v.at[j]], add=True)
        return carry

    lax.fori_loop(0, NCH, body, 0)
    plsc.subcore_barrier()
    w = c * NSUB + s
    rows = NP // (NCORE * NSUB)
    pltpu.sync_copy(acc.at[pl.ds(w * rows, rows)], deg_hbm.at[pl.ds(w * rows, rows)])


@functools.cache
def _make_gs_kernel():
    return functools.partial(
        pl.kernel,
        out_type=jax.ShapeDtypeStruct((NCORE, NP, HALF), jnp.float32),
        mesh=_sc_mesh(),
        scratch_types=[
            pltpu.VMEM((2, KB, CHUNK), jnp.int32),
            pltpu.VMEM((2, KB, CHUNK), jnp.int32),
            pltpu.VMEM((2, CHUNK, HALF), jnp.float32),
            pltpu.VMEM_SHARED((NP, HALF), jnp.float32),
            pltpu.SemaphoreType.DMA,
            pltpu.SemaphoreType.DMA,
            pltpu.SemaphoreType.DMA,
            pltpu.SemaphoreType.DMA,
        ],
    )(_gs_body)


def _gs_body(y_hbm, src_hbm, dst_hbm, zeros_hbm, out_hbm,
             sidx, didx, gbuf, acc, gs0, gs1, isa, isb):
    c = lax.axis_index("c")
    s = lax.axis_index("s")
    gsem = (gs0, gs1)
    isem = (isa, isb)

    def idx_start(bi, p):
        pltpu.async_copy(src_hbm.at[c, s, bi], sidx.at[p], isem[p])
        pltpu.async_copy(dst_hbm.at[s, bi], didx.at[p], isem[p])

    def idx_wait(bi, p):
        pltpu.make_async_copy(src_hbm.at[c, s, bi], sidx.at[p], isem[p]).wait()
        pltpu.make_async_copy(dst_hbm.at[s, bi], didx.at[p], isem[p]).wait()

    def g_start(src_ref, t, gp):
        pltpu.async_copy(y_hbm.at[src_ref.at[t]], gbuf.at[gp], gsem[gp])

    def g_wait(src_ref, t, gp):
        pltpu.make_async_copy(y_hbm.at[src_ref.at[t]], gbuf.at[gp],
                              gsem[gp]).wait()

    def process(ps, pn):
        cs, cd = sidx.at[ps], didx.at[ps]
        for t in range(KB):
            gp = t % 2
            g_wait(cs, t, gp)
            if t + 1 < KB:
                g_start(cs, t + 1, (t + 1) % 2)
            elif pn is not None:
                g_start(sidx.at[pn], 0, 0)
            pltpu.sync_copy(gbuf.at[gp], acc.at[cd.at[t]], add=True)

    pltpu.sync_copy(src_hbm.at[c, s, 0], sidx.at[0])
    pltpu.sync_copy(dst_hbm.at[s, 0], didx.at[0])
    pltpu.sync_copy(zeros_hbm.at[pl.ds(s * RPT, RPT)], acc.at[pl.ds(s * RPT, RPT)])
    plsc.subcore_barrier()
    idx_start(1, 1)
    g_start(sidx.at[0], 0, 0)

    def body(p, carry):
        bi = p * 2
        idx_wait(bi + 1, 1)
        process(0, 1)
        idx_start(bi + 2, 0)
        idx_wait(bi + 2, 0)
        process(1, 0)
        idx_start(bi + 3, 1)
        return carry

    lax.fori_loop(0, NB // 2 - 1, body, 0)
    idx_wait(NB - 1, 1)
    process(0, 1)
    process(1, None)
    plsc.subcore_barrier()
    pltpu.sync_copy(acc.at[pl.ds(s * RPT, RPT)],
                    out_hbm.at[c, pl.ds(s * RPT, RPT)])


def _tc_first_body(x_ref, deg_ref, w_ref, y_ref):
    dis = lax.rsqrt(deg_ref[:, :1] + 1.0)
    y = jnp.dot(x_ref[...], w_ref[...], preferred_element_type=jnp.float32) * dis
    y_ref[0] = y[:, :HALF]
    y_ref[1] = y[:, HALF:]


def _tc_mid_body(acc_ref, y_ref, deg_ref, b_ref, w_ref, out_ref):
    dis = lax.rsqrt(deg_ref[:, :1] + 1.0)
    a = jnp.concatenate([acc_ref[0], acc_ref[1]], axis=-1)
    yv = jnp.concatenate([y_ref[0], y_ref[1]], axis=-1)
    h = jnp.maximum(dis * (a + yv) + b_ref[0, :], 0.0)
    y2 = jnp.dot(h, w_ref[...], preferred_element_type=jnp.float32) * dis
    out_ref[0] = y2[:, :HALF]
    out_ref[1] = y2[:, HALF:]


def _tc_final_body(acc_ref, y_ref, deg_ref, batch_ref, b3_ref, wlin_ref,
                   blin_ref, out_ref, sums_s, cnt_s):
    i = pl.program_id(0)

    @pl.when(i == 0)
    def _():
        sums_s[...] = jnp.zeros_like(sums_s)
        cnt_s[...] = jnp.zeros_like(cnt_s)

    dis = lax.rsqrt(deg_ref[:, :1] + 1.0)
    a = jnp.concatenate([acc_ref[0], acc_ref[1]], axis=-1)
    yv = jnp.concatenate([y_ref[0], y_ref[1]], axis=-1)
    h = dis * (a + yv)
    p = (batch_ref[:, :1] == lax.broadcasted_iota(jnp.int32, (RB, G), 1))
    p = p.astype(jnp.float32)
    sums_s[...] += lax.dot_general(p, h, (((0,), (0,)), ((), ())),
                                   preferred_element_type=jnp.float32)
    cnt_s[...] += lax.dot_general(p, jnp.ones((RB, 128), jnp.float32),
                                  (((0,), (0,)), ((), ())),
                                  preferred_element_type=jnp.float32)

    @pl.when(i == pl.num_programs(0) - 1)
    def _():
        pooled = sums_s[...] / jnp.maximum(cnt_s[:, :1], 1.0) + b3_ref[0, :]
        out_ref[...] = (jnp.dot(pooled, wlin_ref[...],
                                preferred_element_type=jnp.float32)
                        + blin_ref[0, :])


def _tc_first(x_p, deg128, W1):
    return pl.pallas_call(
        _tc_first_body,
        grid=(NP // RB,),
        in_specs=[
            pl.BlockSpec((RB, D_IN), lambda i: (i, 0)),
            pl.BlockSpec((RB, 128), lambda i: (i, 0)),
            pl.BlockSpec((D_IN, D_H), lambda i: (0, 0)),
        ],
        out_specs=pl.BlockSpec((NCORE, RB, HALF), lambda i: (0, i, 0)),
        out_shape=jax.ShapeDtypeStruct((NCORE, NP, HALF), jnp.float32),
    )(x_p, deg128, W1)


def _tc_mid(acc, y, deg128, b, W):
    return pl.pallas_call(
        _tc_mid_body,
        grid=(NP // RB,),
        in_specs=[
            pl.BlockSpec((NCORE, RB, HALF), lambda i: (0, i, 0)),
            pl.BlockSpec((NCORE, RB, HALF), lambda i: (0, i, 0)),
            pl.BlockSpec((RB, 128), lambda i: (i, 0)),
            pl.BlockSpec((1, D_H), lambda i: (0, 0)),
            pl.BlockSpec((D_H, D_H), lambda i: (0, 0)),
        ],
        out_specs=pl.BlockSpec((NCORE, RB, HALF), lambda i: (0, i, 0)),
        out_shape=jax.ShapeDtypeStruct((NCORE, NP, HALF), jnp.float32),
    )(acc, y, deg128, b, W)


def _tc_final(acc, y, deg128, batch_p, b3, Wlin, blin):
    return pl.pallas_call(
        _tc_final_body,
        grid=(NP // RB,),
        in_specs=[
            pl.BlockSpec((NCORE, RB, HALF), lambda i: (0, i, 0)),
            pl.BlockSpec((NCORE, RB, HALF), lambda i: (0, i, 0)),
            pl.BlockSpec((RB, 128), lambda i: (i, 0)),
            pl.BlockSpec((RB, 1), lambda i: (i, 0)),
            pl.BlockSpec((1, D_H), lambda i: (0, 0)),
            pl.BlockSpec((D_H, D_OUT), lambda i: (0, 0)),
            pl.BlockSpec((1, D_OUT), lambda i: (0, 0)),
        ],
        out_specs=pl.BlockSpec((G, D_OUT), lambda i: (0, 0)),
        out_shape=jax.ShapeDtypeStruct((G, D_OUT), jnp.float32),
        scratch_shapes=[
            pltpu.VMEM((G, D_H), jnp.float32),
            pltpu.VMEM((G, 128), jnp.float32),
        ],
    )(acc, y, deg128, batch_p, b3, Wlin, blin)


def kernel(x, edge_index, batch, W1, b1, W2, b2, W3, b3, Wlin, blin):
    i32 = jnp.int32
    f32 = jnp.float32
    src = edge_index[0]
    dst = edge_index[1]
    src_p = jnp.full((EP,), DUMMY, i32).at[:E].set(src)
    dst_p = jnp.full((EP,), DUMMY, i32).at[:E].set(dst)
    src2 = (src_p[None] + (jnp.arange(NCORE, dtype=i32) * NP)[:, None]
            ).reshape(NCORE, NSUB, NB, KB, CHUNK)
    dst_deg = dst_p.reshape(NSUB, NCH, CHUNK)
    dst_gs = dst_p.reshape(NSUB, NB, KB, CHUNK)
    x_p = jnp.zeros((NP, D_IN), f32).at[:N].set(x)
    batch_p = jnp.full((NP, 1), G, i32).at[:N, 0].set(batch)
    zeros_feat = jnp.zeros((NP, HALF), f32)
    ones128 = jnp.ones((CHUNK, 128), f32)

    deg128 = _make_deg_kernel()(dst_deg, ones128, zeros_feat)
    gs = _make_gs_kernel()
    y1 = _tc_first(x_p, deg128, W1)
    acc1 = gs(y1.reshape(NCORE * NP, HALF), src2, dst_gs, zeros_feat)
    y2 = _tc_mid(acc1, y1, deg128, b1.reshape(1, -1), W2)
    acc2 = gs(y2.reshape(NCORE * NP, HALF), src2, dst_gs, zeros_feat)
    y3 = _tc_mid(acc2, y2, deg128, b2.reshape(1, -1), W3)
    acc3 = gs(y3.reshape(NCORE * NP, HALF), src2, dst_gs, zeros_feat)
    return _tc_final(acc3, y3, deg128, batch_p, b3.reshape(1, -1), Wlin,
                     blin.reshape(1, -1))


# speedup vs baseline: 7.4676x; 1.0012x over previous
import functools

import jax
import jax.numpy as jnp
from jax import lax
from jax.experimental import pallas as pl
from jax.experimental.pallas import tpu as pltpu
from jax.experimental.pallas import tpu_sc as plsc

N = 10000
E = 320000
D_IN = 128
D_H = 256
D_OUT = 128
G = 64

NP = 10240
DUMMY = N
NCORE = 2
NSUB = 16
CHUNK = 128
KB = 8
NB = 20
NCH = NB * KB
EP = NSUB * NCH * CHUNK
RPT = NP // NSUB
HALF = D_H // 2
RB = 1024


def _sc_mesh():
    return plsc.VectorSubcoreMesh(
        core_axis_name="c", subcore_axis_name="s", num_cores=NCORE,
        num_subcores=NSUB)


@functools.cache
def _make_deg_kernel():
    return functools.partial(
        pl.kernel,
        out_type=jax.ShapeDtypeStruct((NP, 128), jnp.float32),
        mesh=_sc_mesh(),
        scratch_types=[
            pltpu.VMEM((NCH, CHUNK), jnp.int32),
            pltpu.VMEM((CHUNK, 128), jnp.float32),
            pltpu.VMEM_SHARED((NP, 128), jnp.float32),
        ],
    )(_deg_body)


def _deg_body(dst_hbm, ones_hbm, zeros_hbm, deg_hbm, dst_v, ones_v, acc):
    c = lax.axis_index("c")
    s = lax.axis_index("s")
    pltpu.sync_copy(dst_hbm.at[s], dst_v)
    pltpu.sync_copy(ones_hbm, ones_v)
    pltpu.sync_copy(zeros_hbm.at[pl.ds(s * RPT, RPT)], acc.at[pl.ds(s * RPT, RPT)])
    plsc.subcore_barrier()

    def body(j, carry):
        pltpu.sync_copy(ones_v, acc.at[dst_v.at[j]], add=True)
        return carry

    lax.fori_loop(0, NCH, body, 0)
    plsc.subcore_barrier()
    w = c * NSUB + s
    rows = NP // (NCORE * NSUB)
    pltpu.sync_copy(acc.at[pl.ds(w * rows, rows)], deg_hbm.at[pl.ds(w * rows, rows)])


@functools.cache
def _make_gs_kernel():
    return functools.partial(
        pl.kernel,
        out_type=jax.ShapeDtypeStruct((NCORE, NP, HALF), jnp.float32),
        mesh=_sc_mesh(),
        scratch_types=[
            pltpu.VMEM((2, KB, CHUNK), jnp.int32),
            pltpu.VMEM((2, KB, CHUNK), jnp.int32),
            pltpu.VMEM((2, CHUNK, HALF), jnp.float32),
            pltpu.VMEM_SHARED((NP, HALF), jnp.float32),
            pltpu.SemaphoreType.DMA,
            pltpu.SemaphoreType.DMA,
            pltpu.SemaphoreType.DMA,
            pltpu.SemaphoreType.DMA,
            pltpu.SemaphoreType.DMA,
            pltpu.SemaphoreType.DMA,
        ],
    )(_gs_body)


def _gs_body(y_hbm, src_hbm, dst_hbm, zeros_hbm, out_hbm,
             sidx, didx, gbuf, acc, gs0, gs1, isa, isb, ss0, ss1):
    c = lax.axis_index("c")
    s = lax.axis_index("s")
    gsem = (gs0, gs1)
    isem = (isa, isb)
    ssem = (ss0, ss1)

    def idx_start(bi, p):
        pltpu.async_copy(src_hbm.at[c, s, bi], sidx.at[p], isem[p])
        pltpu.async_copy(dst_hbm.at[s, bi], didx.at[p], isem[p])

    def idx_wait(bi, p):
        pltpu.make_async_copy(src_hbm.at[c, s, bi], sidx.at[p], isem[p]).wait()
        pltpu.make_async_copy(dst_hbm.at[s, bi], didx.at[p], isem[p]).wait()

    def g_start(src_ref, t, gp):
        pltpu.async_copy(y_hbm.at[src_ref.at[t]], gbuf.at[gp], gsem[gp])

    def g_wait(src_ref, t, gp):
        pltpu.make_async_copy(y_hbm.at[src_ref.at[t]], gbuf.at[gp],
                              gsem[gp]).wait()

    def process(ps, pn):
        cs, cd = sidx.at[ps], didx.at[ps]
        pend = [None, None]

        def drain(q):
            if pend[q] is not None:
                pend[q].wait()
                pend[q] = None

        for t in range(KB):
            gp = t % 2
            g_wait(cs, t, gp)
            if t + 1 < KB:
                drain((t + 1) % 2)
                g_start(cs, t + 1, (t + 1) % 2)
            elif pn is not None:
                drain(0)
                g_start(sidx.at[pn], 0, 0)
            pend[gp] = pltpu.async_copy(gbuf.at[gp], acc.at[cd.at[t]],
                                        ssem[gp], add=True)
        drain(0)
        drain(1)

    pltpu.sync_copy(src_hbm.at[c, s, 0], sidx.at[0])
    pltpu.sync_copy(dst_hbm.at[s, 0], didx.at[0])
    pltpu.sync_copy(zeros_hbm.at[pl.ds(s * RPT, RPT)], acc.at[pl.ds(s * RPT, RPT)])
    plsc.subcore_barrier()
    idx_start(1, 1)
    g_start(sidx.at[0], 0, 0)

    def body(p, carry):
        bi = p * 2
        idx_wait(bi + 1, 1)
        process(0, 1)
        idx_start(bi + 2, 0)
        idx_wait(bi + 2, 0)
        process(1, 0)
        idx_start(bi + 3, 1)
        return carry

    lax.fori_loop(0, NB // 2 - 1, body, 0)
    idx_wait(NB - 1, 1)
    process(0, 1)
    process(1, None)
    plsc.subcore_barrier()
    pltpu.sync_copy(acc.at[pl.ds(s * RPT, RPT)],
                    out_hbm.at[c, pl.ds(s * RPT, RPT)])


def _tc_first_body(x_ref, deg_ref, w_ref, y_ref):
    dis = lax.rsqrt(deg_ref[:, :1] + 1.0)
    y = jnp.dot(x_ref[...], w_ref[...], preferred_element_type=jnp.float32) * dis
    y_ref[0] = y[:, :HALF]
    y_ref[1] = y[:, HALF:]


def _tc_mid_body(acc_ref, y_ref, deg_ref, b_ref, w_ref, out_ref):
    dis = lax.rsqrt(deg_ref[:, :1] + 1.0)
    a = jnp.concatenate([acc_ref[0], acc_ref[1]], axis=-1)
    yv = jnp.concatenate([y_ref[0], y_ref[1]], axis=-1)
    h = jnp.maximum(dis * (a + yv) + b_ref[0, :], 0.0)
    y2 = jnp.dot(h, w_ref[...], preferred_element_type=jnp.float32) * dis
    out_ref[0] = y2[:, :HALF]
    out_ref[1] = y2[:, HALF:]


def _tc_final_body(acc_ref, y_ref, deg_ref, batch_ref, b3_ref, wlin_ref,
                   blin_ref, out_ref, sums_s, cnt_s):
    i = pl.program_id(0)

    @pl.when(i == 0)
    def _():
        sums_s[...] = jnp.zeros_like(sums_s)
        cnt_s[...] = jnp.zeros_like(cnt_s)

    dis = lax.rsqrt(deg_ref[:, :1] + 1.0)
    a = jnp.concatenate([acc_ref[0], acc_ref[1]], axis=-1)
    yv = jnp.concatenate([y_ref[0], y_ref[1]], axis=-1)
    h = dis * (a + yv)
    p = (batch_ref[:, :1] == lax.broadcasted_iota(jnp.int32, (RB, G), 1))
    p = p.astype(jnp.float32)
    sums_s[...] += lax.dot_general(p, h, (((0,), (0,)), ((), ())),
                                   preferred_element_type=jnp.float32)
    cnt_s[...] += lax.dot_general(p, jnp.ones((RB, 128), jnp.float32),
                                  (((0,), (0,)), ((), ())),
                                  preferred_element_type=jnp.float32)

    @pl.when(i == pl.num_programs(0) - 1)
    def _():
        pooled = sums_s[...] / jnp.maximum(cnt_s[:, :1], 1.0) + b3_ref[0, :]
        out_ref[...] = (jnp.dot(pooled, wlin_ref[...],
                                preferred_element_type=jnp.float32)
                        + blin_ref[0, :])


def _tc_first(x_p, deg128, W1):
    return pl.pallas_call(
        _tc_first_body,
        grid=(NP // RB,),
        in_specs=[
            pl.BlockSpec((RB, D_IN), lambda i: (i, 0)),
            pl.BlockSpec((RB, 128), lambda i: (i, 0)),
            pl.BlockSpec((D_IN, D_H), lambda i: (0, 0)),
        ],
        out_specs=pl.BlockSpec((NCORE, RB, HALF), lambda i: (0, i, 0)),
        out_shape=jax.ShapeDtypeStruct((NCORE, NP, HALF), jnp.float32),
    )(x_p, deg128, W1)


def _tc_mid(acc, y, deg128, b, W):
    return pl.pallas_call(
        _tc_mid_body,
        grid=(NP // RB,),
        in_specs=[
            pl.BlockSpec((NCORE, RB, HALF), lambda i: (0, i, 0)),
            pl.BlockSpec((NCORE, RB, HALF), lambda i: (0, i, 0)),
            pl.BlockSpec((RB, 128), lambda i: (i, 0)),
            pl.BlockSpec((1, D_H), lambda i: (0, 0)),
            pl.BlockSpec((D_H, D_H), lambda i: (0, 0)),
        ],
        out_specs=pl.BlockSpec((NCORE, RB, HALF), lambda i: (0, i, 0)),
        out_shape=jax.ShapeDtypeStruct((NCORE, NP, HALF), jnp.float32),
    )(acc, y, deg128, b, W)


def _tc_final(acc, y, deg128, batch_p, b3, Wlin, blin):
    return pl.pallas_call(
        _tc_final_body,
        grid=(NP // RB,),
        in_specs=[
            pl.BlockSpec((NCORE, RB, HALF), lambda i: (0, i, 0)),
            pl.BlockSpec((NCORE, RB, HALF), lambda i: (0, i, 0)),
            pl.BlockSpec((RB, 128), lambda i: (i, 0)),
            pl.BlockSpec((RB, 1), lambda i: (i, 0)),
            pl.BlockSpec((1, D_H), lambda i: (0, 0)),
            pl.BlockSpec((D_H, D_OUT), lambda i: (0, 0)),
            pl.BlockSpec((1, D_OUT), lambda i: (0, 0)),
        ],
        out_specs=pl.BlockSpec((G, D_OUT), lambda i: (0, 0)),
        out_shape=jax.ShapeDtypeStruct((G, D_OUT), jnp.float32),
        scratch_shapes=[
            pltpu.VMEM((G, D_H), jnp.float32),
            pltpu.VMEM((G, 128), jnp.float32),
        ],
    )(acc, y, deg128, batch_p, b3, Wlin, blin)


def kernel(x, edge_index, batch, W1, b1, W2, b2, W3, b3, Wlin, blin):
    i32 = jnp.int32
    f32 = jnp.float32
    src = edge_index[0]
    dst = edge_index[1]
    src_p = jnp.full((EP,), DUMMY, i32).at[:E].set(src)
    dst_p = jnp.full((EP,), DUMMY, i32).at[:E].set(dst)
    src2 = (src_p[None] + (jnp.arange(NCORE, dtype=i32) * NP)[:, None]
            ).reshape(NCORE, NSUB, NB, KB, CHUNK)
    dst_deg = dst_p.reshape(NSUB, NCH, CHUNK)
    dst_gs = dst_p.reshape(NSUB, NB, KB, CHUNK)
    x_p = jnp.zeros((NP, D_IN), f32).at[:N].set(x)
    batch_p = jnp.full((NP, 1), G, i32).at[:N, 0].set(batch)
    zeros_feat = jnp.zeros((NP, HALF), f32)
    ones128 = jnp.ones((CHUNK, 128), f32)

    deg128 = _make_deg_kernel()(dst_deg, ones128, zeros_feat)
    gs = _make_gs_kernel()
    y1 = _tc_first(x_p, deg128, W1)
    acc1 = gs(y1.reshape(NCORE * NP, HALF), src2, dst_gs, zeros_feat)
    y2 = _tc_mid(acc1, y1, deg128, b1.reshape(1, -1), W2)
    acc2 = gs(y2.reshape(NCORE * NP, HALF), src2, dst_gs, zeros_feat)
    y3 = _tc_mid(acc2, y2, deg128, b2.reshape(1, -1), W3)
    acc3 = gs(y3.reshape(NCORE * NP, HALF), src2, dst_gs, zeros_feat)
    return _tc_final(acc3, y3, deg128, batch_p, b3.reshape(1, -1), Wlin,
                     blin.reshape(1, -1))
